# baseline (speedup 1.0000x reference)
import functools

import jax
import jax.numpy as jnp
from jax import lax
from jax.experimental import pallas as pl
from jax.experimental.pallas import tpu as pltpu
from jax.experimental.pallas import tpu_sc as plsc

_B = 16384
_N = 11
_D = 32
_V = 1000000
_NC = 2
_NS = 16
_NW = _NC * _NS
_PER_W = _B // _NW
_CH = 128
_NCH = _PER_W // _CH
_LANES = 16
_NCB = _V // 128
_DG = _D // 8
_ROWS_OUT = (_NCB + 1) * _D
_UCB = 32
_NSB = _NCB // _UCB
_NU = _NSB * _DG


def _relinearize_call(uembT, tailTp):
    mesh = plsc.VectorSubcoreMesh(
        core_axis_name="c", subcore_axis_name="s",
        num_cores=_NC, num_subcores=_NS)

    @functools.partial(
        pl.kernel,
        out_type=jax.ShapeDtypeStruct((_ROWS_OUT, 128), jnp.float32),
        mesh=mesh,
        compiler_params=pltpu.CompilerParams(needs_layout_passes=False),
        scratch_types=[
            pltpu.VMEM((8, _UCB * 128), jnp.float32),
            pltpu.VMEM((8, _UCB * 128), jnp.float32),
            pltpu.SemaphoreType.DMA,
            pltpu.SemaphoreType.DMA,
            pltpu.SemaphoreType.DMA,
            pltpu.SemaphoreType.DMA,
        ],
    )
    def body(uembT_hbm, tail_hbm, ulin_hbm, bbuf0, bbuf1, isem0, isem1,
             osem0, osem1):
        wid = lax.axis_index("c") * _NS + lax.axis_index("s")
        nk = jnp.where(wid < _NU - (_NU // _NW) * _NW,
                       _NU // _NW + 1, _NU // _NW)
        bufs = ((bbuf0, isem0, osem0), (bbuf1, isem1, osem1))

        def start_in(k, buf, isem):
            u = jnp.minimum(wid + _NW * k, _NU - 1)
            sb = u // _DG
            dg = u % _DG
            d0 = pl.multiple_of((u % _DG) * 8, 8)
            c0 = pl.multiple_of((u // _DG) * (_UCB * 128), 128)
            cp = pltpu.make_async_copy(
                uembT_hbm.at[pl.ds(d0, 8), pl.ds(c0, _UCB * 128)], buf, isem)
            cp.start()
            return cp

        def drain_outs(buf, osem):
            for t in range(_UCB):
                pltpu.make_async_copy(
                    buf.at[:, pl.ds(t * 128, 128)],
                    ulin_hbm.at[pl.ds(t * 128, 8), :], osem).wait()

        def start_outs(k, buf, osem):
            u = jnp.minimum(wid + _NW * k, _NU - 1)
            row_u = pl.multiple_of((u // _DG) * (_UCB * _D) + (u % _DG) * 8,
                                   8)
            for t in range(_UCB):
                row0 = pl.multiple_of(row_u + t * _D, 8)
                pltpu.make_async_copy(
                    buf.at[:, pl.ds(t * 128, 128)],
                    ulin_hbm.at[pl.ds(row0, 8), :], osem).start()

        cp0 = start_in(0, bbuf0, isem0)

        def unit_body(k2, carry):
            for p in range(2):
                k = k2 * 2 + p
                buf, isem, osem = bufs[p]
                nbuf, nisem, nosem = bufs[1 - p]

                @pl.when(k < nk)
                def _():
                    @pl.when(k + 1 < nk)
                    def _():
                        @pl.when(k >= 1)
                        def _():
                            drain_outs(nbuf, nosem)

                        start_in(k + 1, nbuf, nisem)

                    pltpu.make_async_copy(
                        uembT_hbm.at[pl.ds(0, 8), pl.ds(0, _UCB * 128)],
                        buf, isem).wait()
                    start_outs(k, buf, osem)

                    @pl.when(k + 1 >= nk)
                    def _():
                        @pl.when(k >= 1)
                        def _():
                            drain_outs(nbuf, nosem)
                        drain_outs(buf, osem)
            return carry

        lax.fori_loop(0, (_NU // _NW + 2) // 2, unit_body, 0)

        @pl.when((wid >= 4) & (wid < 8))
        def _():
            cbr = _NSB * _UCB + (wid - 4)
            for dg in range(_DG):
                pltpu.sync_copy(
                    uembT_hbm.at[pl.ds(dg * 8, 8),
                                 pl.ds(cbr * 128, 128)],
                    bbuf0.at[:, pl.ds(dg * 128, 128)])
            for dg in range(_DG):
                row0 = cbr * _D + dg * 8
                pltpu.sync_copy(
                    bbuf0.at[:, pl.ds(dg * 128, 128)],
                    ulin_hbm.at[pl.ds(row0, 8), :])

        @pl.when(wid < _DG)
        def _():
            d0 = pl.multiple_of(wid * 8, 8)
            cp = pltpu.make_async_copy(
                tail_hbm.at[pl.ds(d0, 8), :],
                ulin_hbm.at[pl.ds(_NCB * _D + d0, 8), :], isem0)
            cp.start()
            cp.wait()

    return body(uembT, tailTp)


def _score_call(sess_flat, prods_flat, ulin, pemb):
    mesh = plsc.VectorSubcoreMesh(
        core_axis_name="c", subcore_axis_name="s",
        num_cores=_NC, num_subcores=_NS)

    scr = []
    for _ in range(2):
        scr += [
            pltpu.VMEM((_CH,), jnp.int32),
            pltpu.VMEM((_D * _CH,), jnp.int32),
            pltpu.VMEM((_N * _CH,), jnp.int32),
            pltpu.VMEM((_D, _CH), jnp.float32),
            pltpu.VMEM((_CH * _N, _D), jnp.float32),
            pltpu.VMEM((_CH, _N), jnp.float32),
            pltpu.SemaphoreType.DMA,
            pltpu.SemaphoreType.DMA,
            pltpu.SemaphoreType.DMA,
        ]

    @functools.partial(
        pl.kernel,
        out_type=jax.ShapeDtypeStruct((_B, _N), jnp.float32),
        mesh=mesh,
        compiler_params=pltpu.CompilerParams(
            use_tc_tiling_on_sc=False, needs_layout_passes=False),
        scratch_types=scr,
    )
    def body(sess_hbm, prods_hbm, ulin_hbm, pemb_hbm, out_hbm, *bufs):
        wid = lax.axis_index("c") * _NS + lax.axis_index("s")
        sets = (bufs[0:9], bufs[9:18])

        def stage(c, bset):
            sidx, uidx, pidx, ubufT, prows, _, usem, psem, _ = bset
            gbase = wid * _PER_W + c * _CH
            pltpu.sync_copy(sess_hbm.at[pl.ds(gbase, _CH)], sidx)
            pltpu.sync_copy(
                prods_hbm.at[pl.ds(gbase * _N, _N * _CH)], pidx)
            for g in range(_CH // _LANES):
                sl = sidx[pl.ds(g * _LANES, _LANES)]
                sbase = (sl // 128) * (_D * 128) + sl % 128
                for d in range(_D):
                    uidx[pl.ds(d * _CH + g * _LANES, _LANES)] = (
                        sbase + d * 128)
            for d in range(_D):
                pltpu.make_async_copy(
                    ulin_hbm.at[uidx.at[pl.ds(d * _CH, _CH)]],
                    ubufT.at[d], usem).start()
            for j in range(_N):
                pltpu.make_async_copy(
                    pemb_hbm.at[pidx.at[pl.ds(j * _CH, _CH)]],
                    prows.at[pl.ds(j * _CH, _CH)], psem).start()

        def consume(c, bset):
            sidx, uidx, pidx, ubufT, prows, outv, usem, psem, osem = bset
            gbase = wid * _PER_W + c * _CH
            for d in range(_D):
                pltpu.make_async_copy(
                    ulin_hbm.at[uidx.at[pl.ds(d * _CH, _CH)]],
                    ubufT.at[d], usem).wait()
            for j in range(_N):
                pltpu.make_async_copy(
                    pemb_hbm.at[pidx.at[pl.ds(j * _CH, _CH)]],
                    prows.at[pl.ds(j * _CH, _CH)], psem).wait()

            @pl.when(c >= 2)
            def _():
                pltpu.make_async_copy(
                    outv, out_hbm.at[pl.ds(gbase, _CH)], osem).wait()

            def group_body(g2, carry2):
                for gg in range(2):
                    g = g2 * 2 + gg
                    bvec = g * _LANES + lax.iota(jnp.int32, _LANES)
                    us = [ubufT[d, pl.ds(g * _LANES, _LANES)]
                          for d in range(_D)]
                    for n in range(_N):
                        qvec = bvec * _N + n
                        acc = jnp.zeros((_LANES,), jnp.float32)
                        for d in range(_D):
                            pv = plsc.load_gather(
                                prows,
                                [qvec, jnp.full((_LANES,), d, jnp.int32)])
                            acc = acc + us[d] * pv
                        plsc.store_scatter(
                            outv, [bvec, jnp.full((_LANES,), n, jnp.int32)],
                            acc)
                return carry2

            lax.fori_loop(0, _CH // _LANES // 2, group_body, 0)
            pltpu.make_async_copy(
                outv, out_hbm.at[pl.ds(gbase, _CH)], osem).start()

        stage(0, sets[0])

        def pair_body(k2, carry):
            for p in range(2):
                c = k2 * 2 + p

                @pl.when(c + 1 < _NCH)
                def _():
                    stage(c + 1, sets[1 - p])

                consume(c, sets[p])
            return carry

        lax.fori_loop(0, _NCH // 2, pair_body, 0)
        for p in range(2):
            outv, osem = sets[p][5], sets[p][8]
            gbase = wid * _PER_W + (_NCH - 2 + p) * _CH
            pltpu.make_async_copy(
                outv, out_hbm.at[pl.ds(gbase, _CH)], osem).wait()

    return body(sess_flat, prods_flat, ulin, pemb)


def kernel(session, products, user_embedding, product_embedding):
    sess_flat = session.reshape(-1)
    prods_flat = products.reshape(-1)
    tailTp = jnp.pad(user_embedding[_NCB * 128:].T, ((0, 0), (0, 64)))
    ulin2d = _relinearize_call(user_embedding.T, tailTp)
    ulin = ulin2d.reshape(-1)
    return _score_call(sess_flat, prods_flat, ulin, product_embedding)


# speedup vs baseline: 1.0182x; 1.0182x over previous
import functools

import jax
import jax.numpy as jnp
from jax import lax
from jax.experimental import pallas as pl
from jax.experimental.pallas import tpu as pltpu
from jax.experimental.pallas import tpu_sc as plsc

_B = 16384
_N = 11
_D = 32
_V = 1000000
_NC = 2
_NS = 16
_NW = _NC * _NS
_PER_W = _B // _NW
_CH = 128
_NCH = _PER_W // _CH
_LANES = 16
_NCB = _V // 128
_DG = _D // 8
_ROWS_OUT = (_NCB + 1) * _D
_UCB = 32
_NSB = _NCB // _UCB
_NU = _NSB * _DG


def _relinearize_call(uembT, tailTp):
    mesh = plsc.VectorSubcoreMesh(
        core_axis_name="c", subcore_axis_name="s",
        num_cores=_NC, num_subcores=_NS)

    @functools.partial(
        pl.kernel,
        out_type=jax.ShapeDtypeStruct((_ROWS_OUT, 128), jnp.float32),
        mesh=mesh,
        compiler_params=pltpu.CompilerParams(needs_layout_passes=False),
        scratch_types=[
            pltpu.VMEM((8, _UCB * 128), jnp.float32),
            pltpu.VMEM((8, _UCB * 128), jnp.float32),
            pltpu.SemaphoreType.DMA,
            pltpu.SemaphoreType.DMA,
            pltpu.SemaphoreType.DMA,
            pltpu.SemaphoreType.DMA,
        ],
    )
    def body(uembT_hbm, tail_hbm, ulin_hbm, bbuf0, bbuf1, isem0, isem1,
             osem0, osem1):
        wid = lax.axis_index("c") * _NS + lax.axis_index("s")
        nk = jnp.where(wid < _NU - (_NU // _NW) * _NW,
                       _NU // _NW + 1, _NU // _NW)
        bufs = ((bbuf0, isem0, osem0), (bbuf1, isem1, osem1))

        def start_in(k, buf, isem):
            u = jnp.minimum(wid + _NW * k, _NU - 1)
            sb = u // _DG
            dg = u % _DG
            d0 = pl.multiple_of((u % _DG) * 8, 8)
            c0 = pl.multiple_of((u // _DG) * (_UCB * 128), 128)
            cp = pltpu.make_async_copy(
                uembT_hbm.at[pl.ds(d0, 8), pl.ds(c0, _UCB * 128)], buf, isem)
            cp.start()
            return cp

        def drain_outs(buf, osem):
            for t in range(_UCB):
                pltpu.make_async_copy(
                    buf.at[:, pl.ds(t * 128, 128)],
                    ulin_hbm.at[pl.ds(t * 128, 8), :], osem).wait()

        def start_outs(k, buf, osem):
            u = jnp.minimum(wid + _NW * k, _NU - 1)
            row_u = pl.multiple_of((u // _DG) * (_UCB * _D) + (u % _DG) * 8,
                                   8)
            for t in range(_UCB):
                row0 = pl.multiple_of(row_u + t * _D, 8)
                pltpu.make_async_copy(
                    buf.at[:, pl.ds(t * 128, 128)],
                    ulin_hbm.at[pl.ds(row0, 8), :], osem).start()

        cp0 = start_in(0, bbuf0, isem0)

        def unit_body(k2, carry):
            for p in range(2):
                k = k2 * 2 + p
                buf, isem, osem = bufs[p]
                nbuf, nisem, nosem = bufs[1 - p]

                @pl.when(k < nk)
                def _():
                    @pl.when(k + 1 < nk)
                    def _():
                        @pl.when(k >= 1)
                        def _():
                            drain_outs(nbuf, nosem)

                        start_in(k + 1, nbuf, nisem)

                    pltpu.make_async_copy(
                        uembT_hbm.at[pl.ds(0, 8), pl.ds(0, _UCB * 128)],
                        buf, isem).wait()
                    start_outs(k, buf, osem)

                    @pl.when(k + 1 >= nk)
                    def _():
                        @pl.when(k >= 1)
                        def _():
                            drain_outs(nbuf, nosem)
                        drain_outs(buf, osem)
            return carry

        lax.fori_loop(0, (_NU // _NW + 2) // 2, unit_body, 0)

        @pl.when((wid >= 4) & (wid < 8))
        def _():
            cbr = _NSB * _UCB + (wid - 4)
            for dg in range(_DG):
                pltpu.sync_copy(
                    uembT_hbm.at[pl.ds(dg * 8, 8),
                                 pl.ds(cbr * 128, 128)],
                    bbuf0.at[:, pl.ds(dg * 128, 128)])
            for dg in range(_DG):
                row0 = cbr * _D + dg * 8
                pltpu.sync_copy(
                    bbuf0.at[:, pl.ds(dg * 128, 128)],
                    ulin_hbm.at[pl.ds(row0, 8), :])

        @pl.when(wid < _DG)
        def _():
            d0 = pl.multiple_of(wid * 8, 8)
            cp = pltpu.make_async_copy(
                tail_hbm.at[pl.ds(d0, 8), :],
                ulin_hbm.at[pl.ds(_NCB * _D + d0, 8), :], isem0)
            cp.start()
            cp.wait()

    return body(uembT, tailTp)


def _score_call(sess_flat, prods_flat, ulin, pemb):
    mesh = plsc.VectorSubcoreMesh(
        core_axis_name="c", subcore_axis_name="s",
        num_cores=_NC, num_subcores=_NS)

    scr = []
    for _ in range(2):
        scr += [
            pltpu.VMEM((_CH,), jnp.int32),
            pltpu.VMEM((_D * _CH,), jnp.int32),
            pltpu.VMEM((_N * _CH,), jnp.int32),
            pltpu.VMEM((_D, _CH), jnp.float32),
            pltpu.VMEM((_CH * _N, _D), jnp.float32),
            pltpu.VMEM((_CH, _N), jnp.float32),
            pltpu.SemaphoreType.DMA,
            pltpu.SemaphoreType.DMA,
            pltpu.SemaphoreType.DMA,
        ]

    @functools.partial(
        pl.kernel,
        out_type=jax.ShapeDtypeStruct((_B, _N), jnp.float32),
        mesh=mesh,
        compiler_params=pltpu.CompilerParams(
            use_tc_tiling_on_sc=False, needs_layout_passes=False),
        scratch_types=scr,
    )
    def body(sess_hbm, prods_hbm, ulin_hbm, pemb_hbm, out_hbm, *bufs):
        wid = lax.axis_index("c") * _NS + lax.axis_index("s")
        sets = (bufs[0:9], bufs[9:18])

        def stage(c, bset):
            sidx, uidx, pidx, ubufT, prows, _, usem, psem, _ = bset
            gbase = wid * _PER_W + c * _CH
            pltpu.sync_copy(sess_hbm.at[pl.ds(gbase, _CH)], sidx)
            pltpu.sync_copy(
                prods_hbm.at[pl.ds(gbase * _N, _N * _CH)], pidx)
            for g in range(_CH // _LANES):
                sl = sidx[pl.ds(g * _LANES, _LANES)]
                sbase = (sl // 128) * (_D * 128) + sl % 128
                for d in range(_D):
                    uidx[pl.ds(d * _CH + g * _LANES, _LANES)] = (
                        sbase + d * 128)
            for d in range(_D):
                pltpu.make_async_copy(
                    ulin_hbm.at[uidx.at[pl.ds(d * _CH, _CH)]],
                    ubufT.at[d], usem).start()
            for j in range(_N):
                pltpu.make_async_copy(
                    pemb_hbm.at[pidx.at[pl.ds(j * _CH, _CH)]],
                    prows.at[pl.ds(j * _CH, _CH)], psem).start()

        def consume(c, bset):
            sidx, uidx, pidx, ubufT, prows, outv, usem, psem, osem = bset
            gbase = wid * _PER_W + c * _CH
            for d in range(_D):
                pltpu.make_async_copy(
                    ulin_hbm.at[uidx.at[pl.ds(d * _CH, _CH)]],
                    ubufT.at[d], usem).wait()
            for j in range(_N):
                pltpu.make_async_copy(
                    pemb_hbm.at[pidx.at[pl.ds(j * _CH, _CH)]],
                    prows.at[pl.ds(j * _CH, _CH)], psem).wait()

            @pl.when(c >= 2)
            def _():
                pltpu.make_async_copy(
                    outv, out_hbm.at[pl.ds(gbase, _CH)], osem).wait()

            def group_body(g, carry2):
                bvec = g * _LANES + lax.iota(jnp.int32, _LANES)
                us = [ubufT[d, pl.ds(g * _LANES, _LANES)] for d in range(_D)]
                for n in range(_N):
                    qvec = bvec * _N + n
                    acc = jnp.zeros((_LANES,), jnp.float32)
                    for d in range(_D):
                        pv = plsc.load_gather(
                            prows, [qvec, jnp.full((_LANES,), d, jnp.int32)])
                        acc = acc + us[d] * pv
                    plsc.store_scatter(
                        outv, [bvec, jnp.full((_LANES,), n, jnp.int32)], acc)
                return carry2

            lax.fori_loop(0, _CH // _LANES, group_body, 0)
            pltpu.make_async_copy(
                outv, out_hbm.at[pl.ds(gbase, _CH)], osem).start()

        stage(0, sets[0])

        def pair_body(k2, carry):
            for p in range(2):
                c = k2 * 2 + p

                @pl.when(c + 1 < _NCH)
                def _():
                    stage(c + 1, sets[1 - p])

                consume(c, sets[p])
            return carry

        lax.fori_loop(0, _NCH // 2, pair_body, 0)
        for p in range(2):
            outv, osem = sets[p][5], sets[p][8]
            gbase = wid * _PER_W + (_NCH - 2 + p) * _CH
            pltpu.make_async_copy(
                outv, out_hbm.at[pl.ds(gbase, _CH)], osem).wait()

    return body(sess_flat, prods_flat, ulin, pemb)


def kernel(session, products, user_embedding, product_embedding):
    sess_flat = session.reshape(-1)
    prods_flat = products.reshape(-1)
    tailTp = jnp.pad(user_embedding[_NCB * 128:].T, ((0, 0), (0, 64)))
    ulin2d = _relinearize_call(user_embedding.T, tailTp)
    ulin = ulin2d.reshape(-1)
    return _score_call(sess_flat, prods_flat, ulin, product_embedding)
